# baseline (speedup 1.0000x reference)
import jax
import jax.numpy as jnp
from jax.experimental import pallas as pl
import jax.experimental.pallas.tpu as pltpu

_NUM_IDX = 1000
_PAD = 1024
_RB = 512
_BB = 4


def _dd_kernel(sigma_ref, w_ref, sigmas_ref, x_ref, o_ref, a_sm, bias_sm):
    b = pl.program_id(0)

    @pl.when(b == 0)
    def _compute_scalars():
        sig = sigma_ref[:, :]
        sp = sigmas_ref[:, :]
        dist = jnp.abs(sig - sp)
        minval = jnp.min(dist, axis=1, keepdims=True)
        lin = jax.lax.broadcasted_iota(jnp.int32, (16, _PAD), 1)
        idx = jnp.min(
            jnp.where(dist == minval, lin, jnp.int32(1 << 30)),
            axis=1, keepdims=True,
        )
        spb = jnp.broadcast_to(sp, (16, _PAD))
        sq = jnp.sum(jnp.where(lin == idx, spb, 0.0), axis=1, keepdims=True)
        a = -sq / jnp.sqrt(sq * sq + 1.0)
        bias = -sq * (idx.astype(jnp.float32) / _NUM_IDX)
        for i in range(16):
            a_sm[i] = a[i, 0]
            bias_sm[i] = bias[i, 0]

    ch = 64
    for bb in range(_BB):
        a_b = a_sm[b * _BB + bb]
        bias = bias_sm[b * _BB + bb]
        aw = [[a_b * w_ref[c, d] + (1.0 if c == d else 0.0) for d in range(3)]
              for c in range(3)]

        def body(i, carry, bb=bb, aw=aw, bias=bias):
            r = pl.multiple_of(i * ch, ch)
            x0 = x_ref[bb, 0, pl.ds(r, ch), :]
            x1 = x_ref[bb, 1, pl.ds(r, ch), :]
            x2 = x_ref[bb, 2, pl.ds(r, ch), :]
            for d in range(3):
                o_ref[bb, d, pl.ds(r, ch), :] = (
                    aw[0][d] * x0 + aw[1][d] * x1 + aw[2][d] * x2 + bias
                )
            return carry

        jax.lax.fori_loop(0, _RB // ch, body, 0)


@jax.jit
def kernel(inputs, sigma, W, sigmas):
    B, C, H, Wd = inputs.shape
    sigmas_p = jnp.concatenate(
        [sigmas, jnp.full((_PAD - _NUM_IDX,), 1e30, dtype=sigmas.dtype)]
    ).reshape(1, _PAD)
    return pl.pallas_call(
        _dd_kernel,
        grid=(B // _BB,),
        in_specs=[
            pl.BlockSpec((B, 1), lambda b: (0, 0)),
            pl.BlockSpec(memory_space=pltpu.SMEM),
            pl.BlockSpec((1, _PAD), lambda b: (0, 0)),
            pl.BlockSpec((_BB, C, _RB, Wd), lambda b: (b, 0, 0, 0)),
        ],
        out_specs=pl.BlockSpec((_BB, C, _RB, Wd), lambda b: (b, 0, 0, 0)),
        out_shape=jax.ShapeDtypeStruct((B, C, H, Wd), inputs.dtype),
        scratch_shapes=[
            pltpu.SMEM((B,), jnp.float32),
            pltpu.SMEM((B,), jnp.float32),
        ],
        compiler_params=pltpu.CompilerParams(
            dimension_semantics=("arbitrary",),
        ),
    )(sigma.reshape(B, 1), W, sigmas_p, inputs)


# speedup vs baseline: 1.0100x; 1.0100x over previous
import jax
import jax.numpy as jnp
from jax.experimental import pallas as pl
import jax.experimental.pallas.tpu as pltpu

_NUM_IDX = 1000
_PAD = 1024
_RB = 512
_BB = 4


def _dd_kernel(sigma_ref, w_ref, sigmas_ref, x_ref, o_ref, a_sm, bias_sm):
    b = pl.program_id(0)

    @pl.when(b == 0)
    def _compute_scalars():
        sig = sigma_ref[:, :]
        sp = sigmas_ref[:, :]
        dist = jnp.abs(sig - sp)
        minval = jnp.min(dist, axis=1, keepdims=True)
        lin = jax.lax.broadcasted_iota(jnp.int32, (16, _PAD), 1)
        idx = jnp.min(
            jnp.where(dist == minval, lin, jnp.int32(1 << 30)),
            axis=1, keepdims=True,
        )
        spb = jnp.broadcast_to(sp, (16, _PAD))
        sq = jnp.sum(jnp.where(lin == idx, spb, 0.0), axis=1, keepdims=True)
        a = -sq / jnp.sqrt(sq * sq + 1.0)
        bias = -sq * (idx.astype(jnp.float32) / _NUM_IDX)
        for i in range(16):
            a_sm[i] = a[i, 0]
            bias_sm[i] = bias[i, 0]

    ch = 32
    aws = []
    biases = []
    for bb in range(_BB):
        a_b = a_sm[b * _BB + bb]
        biases.append(bias_sm[b * _BB + bb])
        aws.append([[a_b * w_ref[c, d] + (1.0 if c == d else 0.0)
                     for d in range(3)] for c in range(3)])

    def body(i, carry):
        r = pl.multiple_of(i * ch, ch)
        for bb in range(_BB):
            aw = aws[bb]
            x0 = x_ref[bb, 0, pl.ds(r, ch), :]
            x1 = x_ref[bb, 1, pl.ds(r, ch), :]
            x2 = x_ref[bb, 2, pl.ds(r, ch), :]
            for d in range(3):
                o_ref[bb, d, pl.ds(r, ch), :] = (
                    aw[0][d] * x0 + aw[1][d] * x1 + aw[2][d] * x2 + biases[bb]
                )
        return carry

    jax.lax.fori_loop(0, _RB // ch, body, 0)


@jax.jit
def kernel(inputs, sigma, W, sigmas):
    B, C, H, Wd = inputs.shape
    sigmas_p = jnp.concatenate(
        [sigmas, jnp.full((_PAD - _NUM_IDX,), 1e30, dtype=sigmas.dtype)]
    ).reshape(1, _PAD)
    return pl.pallas_call(
        _dd_kernel,
        grid=(B // _BB,),
        in_specs=[
            pl.BlockSpec((B, 1), lambda b: (0, 0)),
            pl.BlockSpec(memory_space=pltpu.SMEM),
            pl.BlockSpec((1, _PAD), lambda b: (0, 0)),
            pl.BlockSpec((_BB, C, _RB, Wd), lambda b: (b, 0, 0, 0)),
        ],
        out_specs=pl.BlockSpec((_BB, C, _RB, Wd), lambda b: (b, 0, 0, 0)),
        out_shape=jax.ShapeDtypeStruct((B, C, H, Wd), inputs.dtype),
        scratch_shapes=[
            pltpu.SMEM((B,), jnp.float32),
            pltpu.SMEM((B,), jnp.float32),
        ],
        compiler_params=pltpu.CompilerParams(
            dimension_semantics=("arbitrary",),
        ),
    )(sigma.reshape(B, 1), W, sigmas_p, inputs)
